# baseline (speedup 1.0000x reference)
import dataclasses
import functools

import jax
import jax.numpy as jnp
from jax import lax
from jax.experimental import pallas as pl
from jax.experimental.pallas import tpu as pltpu
from jax.experimental.pallas import tpu_sc as plsc

N = 50000
D = 128
E = 400000
E_ROWS = 3200
E_PAD = E_ROWS * 128
RANGE = 8448
PER_TILE = 528
N_RANGES = 6
N_PAD = N_RANGES * RANGE
CHUNK_ROWS = 8
TILE_EDGE_ROWS = E_ROWS // 16


CROWS = RANGE // 128
CROWS_PAD = 72

_SC_PARAMS = pltpu.CompilerParams()
if "needs_layout_passes" in pltpu.CompilerParams.__dataclass_fields__:
    _SC_PARAMS = dataclasses.replace(_SC_PARAMS, needs_layout_passes=False)


def _seg_sum(x_src, src2d, dst2d, zeros_mat, iden_row):
    @functools.partial(
        pl.kernel,
        out_type=(jax.ShapeDtypeStruct((N_PAD, D), jnp.float32),
                  jax.ShapeDtypeStruct((N_RANGES * CROWS_PAD, 128),
                                       jnp.float32)),
        mesh=plsc.VectorSubcoreMesh(core_axis_name="c", subcore_axis_name="s"),
        compiler_params=_SC_PARAMS,
        scratch_types=[
            pltpu.VMEM((CHUNK_ROWS, 128), jnp.int32),
            pltpu.VMEM((CHUNK_ROWS, 128), jnp.int32),
            pltpu.VMEM((CHUNK_ROWS, 128), jnp.int32),
            pltpu.VMEM((128, D), jnp.float32),
            pltpu.VMEM((CROWS_PAD, 128), jnp.float32),
            pltpu.VMEM((1, CROWS_PAD), jnp.int32),
            pltpu.VMEM_SHARED((RANGE + 1, D), jnp.float32),
            pltpu.VMEM_SHARED((CROWS_PAD, 128), jnp.float32),
        ],
    )
    def k(x_hbm, src_hbm, dst_hbm, zm_hbm, iden_hbm,
          sum_out, cnt_out, src_v, dst_v, lds_v, rows_v, cnt_v, iden_v,
          acc, cacc):
        c = lax.axis_index("c")
        t = lax.axis_index("s")
        pltpu.sync_copy(iden_hbm, iden_v)
        ones16 = jnp.full((16,), 1.0, jnp.float32)
        for p in range(3):
            rid = 3 * c + p
            lo = rid * RANGE
            pltpu.sync_copy(zm_hbm, acc.at[pl.ds(t * PER_TILE, PER_TILE), :])
            pltpu.sync_copy(zm_hbm.at[pl.ds(0, CROWS_PAD), :], cnt_v)

            @pl.when(t == 0)
            def _():
                pltpu.sync_copy(zm_hbm.at[pl.ds(0, CROWS_PAD), :], cacc)

            plsc.subcore_barrier()

            @pl.loop(0, TILE_EDGE_ROWS, step=CHUNK_ROWS)
            def _(i):
                row = t * TILE_EDGE_ROWS + i
                pltpu.sync_copy(src_hbm.at[pl.ds(row, CHUNK_ROWS), :], src_v)
                pltpu.sync_copy(dst_hbm.at[pl.ds(row, CHUNK_ROWS), :], dst_v)
                for r in range(CHUNK_ROWS):
                    for j in range(8):
                        sl = pl.ds(j * 16, 16)
                        d = dst_v[r, sl]
                        ld = d - lo
                        ok = (ld >= 0) & (ld < RANGE)
                        ld = jnp.where(ok, ld, RANGE)
                        lds_v[r, sl] = ld
                        plsc.addupdate_scatter(
                            cnt_v,
                            [lax.shift_right_logical(ld, 7), ld & 127],
                            ones16)
                for r in range(CHUNK_ROWS):
                    pltpu.sync_copy(x_hbm.at[src_v.at[r]], rows_v)
                    pltpu.sync_copy(rows_v, acc.at[lds_v.at[r]], add=True)

            pltpu.sync_copy(cnt_v, cacc.at[iden_v.at[0]], add=True)
            plsc.subcore_barrier()
            pltpu.sync_copy(acc.at[pl.ds(t * PER_TILE, PER_TILE), :],
                            sum_out.at[pl.ds(lo + t * PER_TILE, PER_TILE), :])

            @pl.when(t == 0)
            def _():
                pltpu.sync_copy(
                    cacc, cnt_out.at[pl.ds(rid * CROWS_PAD, CROWS_PAD), :])

            plsc.subcore_barrier()

    return k(x_src, src2d, dst2d, zeros_mat, iden_row)


BR = 400


def _dense(summed, cnt, x_dst, Wl, bl, Wr):
    def body(s_ref, c_ref, xd_ref, wl_ref, bl_ref, wr_ref, o_ref):
        mean = s_ref[...] / jnp.maximum(c_ref[...], 1.0)
        out = (jnp.dot(mean, wl_ref[...], preferred_element_type=jnp.float32)
               + bl_ref[...]
               + jnp.dot(xd_ref[...], wr_ref[...],
                         preferred_element_type=jnp.float32))
        nrm = jnp.sqrt(jnp.sum(out * out, axis=1, keepdims=True))
        out = out / jnp.maximum(nrm, 1e-12)
        o_ref[...] = jnp.maximum(out, 0.0)

    return pl.pallas_call(
        body,
        grid=(N // BR,),
        in_specs=[pl.BlockSpec((BR, D), lambda i: (i, 0)),
                  pl.BlockSpec((BR, 1), lambda i: (i, 0)),
                  pl.BlockSpec((BR, D), lambda i: (i, 0)),
                  pl.BlockSpec((D, D), lambda i: (0, 0)),
                  pl.BlockSpec((1, D), lambda i: (0, 0)),
                  pl.BlockSpec((D, D), lambda i: (0, 0))],
        out_specs=pl.BlockSpec((BR, D), lambda i: (i, 0)),
        out_shape=jax.ShapeDtypeStruct((N, D), jnp.float32),
    )(summed, cnt, x_dst, Wl, bl.reshape(1, D), Wr)


def kernel(x_user, x_item, edge_index_user_to_item, edge_index_item_to_user,
           Wl1_u2i, bl1_u2i, Wr1_u2i, Wl1_i2u, bl1_i2u, Wr1_i2u,
           Wl2_u2i, bl2_u2i, Wr2_u2i, Wl2_i2u, bl2_i2u, Wr2_i2u):
    def prep(ei):
        src = ei[0].astype(jnp.int32)
        dst = ei[1].astype(jnp.int32)
        pad = E_PAD - src.shape[0]
        src2d = jnp.concatenate(
            [src, jnp.zeros((pad,), jnp.int32)]).reshape(E_ROWS, 128)
        dst2d = jnp.concatenate(
            [dst, jnp.full((pad,), 2 ** 30, jnp.int32)]).reshape(E_ROWS, 128)
        return src2d, dst2d

    su, du = prep(edge_index_user_to_item)
    si, di = prep(edge_index_item_to_user)
    zm = jnp.zeros((PER_TILE, D), jnp.float32)
    iden = jnp.arange(CROWS_PAD, dtype=jnp.int32).reshape(1, CROWS_PAD)

    def sage(x_src, x_dst, s2, d2, Wl, bl, Wr):
        sm, ct = _seg_sum(x_src, s2, d2, zm, iden)
        ct = ct.reshape(N_RANGES, CROWS_PAD, 128)[:, :CROWS, :]
        ct = ct.reshape(N_PAD)[:N]
        return _dense(sm[:N], ct.reshape(N, 1), x_dst, Wl, bl, Wr)

    item1 = sage(x_user, x_item, su, du, Wl1_u2i, bl1_u2i, Wr1_u2i)
    user1 = sage(x_item, x_user, si, di, Wl1_i2u, bl1_i2u, Wr1_i2u)
    item2 = sage(user1, item1, su, du, Wl2_u2i, bl2_u2i, Wr2_u2i)
    user2 = sage(item1, user1, si, di, Wl2_i2u, bl2_i2u, Wr2_i2u)
    return (user2, item2)


# speedup vs baseline: 4.6652x; 4.6652x over previous
import dataclasses
import functools

import jax
import jax.numpy as jnp
from jax import lax
from jax.experimental import pallas as pl
from jax.experimental.pallas import tpu as pltpu
from jax.experimental.pallas import tpu_sc as plsc

N = 50000
D = 128
E = 400000
E_ROWS = 3200
E_PAD = E_ROWS * 128
RANGE = 8448
PER_TILE = 528
N_RANGES = 6
N_PAD = N_RANGES * RANGE
CHUNK_ROWS = 8
TILE_EDGE_ROWS = E_ROWS // 16
CAP = 1664


CROWS = RANGE // 128
CROWS_PAD = 72

_SC_PARAMS = pltpu.CompilerParams()
if "needs_layout_passes" in pltpu.CompilerParams.__dataclass_fields__:
    _SC_PARAMS = dataclasses.replace(_SC_PARAMS, needs_layout_passes=False)


def _seg_sum(x_src, src2d, dst2d, zeros_mat, iden_row):
    @functools.partial(
        pl.kernel,
        out_type=(jax.ShapeDtypeStruct((N_PAD, D), jnp.float32),
                  jax.ShapeDtypeStruct((N_RANGES * CROWS_PAD, 128),
                                       jnp.float32)),
        mesh=plsc.VectorSubcoreMesh(core_axis_name="c", subcore_axis_name="s"),
        compiler_params=_SC_PARAMS,
        scratch_types=[
            pltpu.VMEM((CHUNK_ROWS, 128), jnp.int32),
            pltpu.VMEM((CHUNK_ROWS, 128), jnp.int32),
            pltpu.VMEM((CAP,), jnp.int32),
            pltpu.VMEM((CAP,), jnp.int32),
            pltpu.VMEM((1, 128), jnp.int32),
            pltpu.VMEM((128, D), jnp.float32),
            pltpu.VMEM((CROWS_PAD, 128), jnp.float32),
            pltpu.VMEM((1, CROWS_PAD), jnp.int32),
            pltpu.VMEM_SHARED((RANGE + 1, D), jnp.float32),
            pltpu.VMEM_SHARED((CROWS_PAD, 128), jnp.float32),
        ],
    )
    def k(x_hbm, src_hbm, dst_hbm, zm_hbm, iden_hbm,
          sum_out, cnt_out, src_v, dst_v, csrc, cdst, stage, rows_v, cnt_v,
          iden_v, acc, cacc):
        c = lax.axis_index("c")
        t = lax.axis_index("s")
        pltpu.sync_copy(iden_hbm, iden_v)
        ones16 = jnp.full((16,), 1.0, jnp.float32)
        tmask = jnp.full((16,), True)
        sent_d = jnp.full((16,), RANGE, jnp.int32)
        sent_s = jnp.zeros((16,), jnp.int32)
        for p in range(3):
            rid = 3 * c + p
            lo = rid * RANGE
            pltpu.sync_copy(zm_hbm, acc.at[pl.ds(t * PER_TILE, PER_TILE), :])
            pltpu.sync_copy(zm_hbm.at[pl.ds(0, CROWS_PAD), :], cnt_v)

            @pl.when(t == 0)
            def _():
                pltpu.sync_copy(zm_hbm.at[pl.ds(0, CROWS_PAD), :], cacc)

            plsc.subcore_barrier()

            def flush(nfl):
                def gs_body(b, carry):
                    for kk in range(8):
                        stage[0, pl.ds(kk * 16, 16)] = (
                            cdst[pl.ds(b * 128 + kk * 16, 16)])
                    pltpu.sync_copy(x_hbm.at[csrc.at[pl.ds(b * 128, 128)]],
                                    rows_v)
                    pltpu.sync_copy(rows_v, acc.at[stage.at[0]], add=True)
                    return carry

                lax.fori_loop(0, nfl, gs_body, jnp.int32(0))

            def chunk_body(ci, off):
                row = t * TILE_EDGE_ROWS + ci * CHUNK_ROWS
                pltpu.sync_copy(src_hbm.at[pl.ds(row, CHUNK_ROWS), :], src_v)
                pltpu.sync_copy(dst_hbm.at[pl.ds(row, CHUNK_ROWS), :], dst_v)
                for r in range(CHUNK_ROWS):
                    for j in range(8):
                        sl = pl.ds(j * 16, 16)
                        s = src_v[r, sl]
                        d = dst_v[r, sl]
                        ld = d - lo
                        ok = (ld >= 0) & (ld < RANGE)
                        ldc = jnp.where(ok, ld, RANGE)
                        plsc.addupdate_scatter(
                            cnt_v,
                            [lax.shift_right_logical(ldc, 7), ldc & 127],
                            ones16)
                        plsc.store_compressed(csrc.at[pl.ds(off, 16)], s,
                                              mask=ok)
                        plsc.store_compressed(cdst.at[pl.ds(off, 16)], ld,
                                              mask=ok)
                        npc = plsc.all_reduce_population_count(ok)
                        off = off + lax.reduce_max(npc, (0,))
                nfl = lax.shift_right_logical(off, 7)
                flush(nfl)
                base = nfl * 128
                for kk in range(8):
                    ts_ = csrc[pl.ds(base + kk * 16, 16)]
                    td_ = cdst[pl.ds(base + kk * 16, 16)]
                    csrc[pl.ds(kk * 16, 16)] = ts_
                    cdst[pl.ds(kk * 16, 16)] = td_
                return off & 127

            off = lax.fori_loop(0, TILE_EDGE_ROWS // CHUNK_ROWS, chunk_body,
                                jnp.int32(0))
            for kk in range(8):
                plsc.store_compressed(cdst.at[pl.ds(off + kk * 16, 16)],
                                      sent_d, mask=tmask)
                plsc.store_compressed(csrc.at[pl.ds(off + kk * 16, 16)],
                                      sent_s, mask=tmask)
            flush(lax.shift_right_logical(off + 127, 7))

            pltpu.sync_copy(cnt_v, cacc.at[iden_v.at[0]], add=True)
            plsc.subcore_barrier()
            pltpu.sync_copy(acc.at[pl.ds(t * PER_TILE, PER_TILE), :],
                            sum_out.at[pl.ds(lo + t * PER_TILE, PER_TILE), :])

            @pl.when(t == 0)
            def _():
                pltpu.sync_copy(
                    cacc, cnt_out.at[pl.ds(rid * CROWS_PAD, CROWS_PAD), :])

            plsc.subcore_barrier()

    return k(x_src, src2d, dst2d, zeros_mat, iden_row)


BR = 400


def _dense(summed, cnt, x_dst, Wl, bl, Wr):
    def body(s_ref, c_ref, xd_ref, wl_ref, bl_ref, wr_ref, o_ref):
        mean = s_ref[...] / jnp.maximum(c_ref[...], 1.0)
        out = (jnp.dot(mean, wl_ref[...], preferred_element_type=jnp.float32)
               + bl_ref[...]
               + jnp.dot(xd_ref[...], wr_ref[...],
                         preferred_element_type=jnp.float32))
        nrm = jnp.sqrt(jnp.sum(out * out, axis=1, keepdims=True))
        out = out / jnp.maximum(nrm, 1e-12)
        o_ref[...] = jnp.maximum(out, 0.0)

    return pl.pallas_call(
        body,
        grid=(N // BR,),
        in_specs=[pl.BlockSpec((BR, D), lambda i: (i, 0)),
                  pl.BlockSpec((BR, 1), lambda i: (i, 0)),
                  pl.BlockSpec((BR, D), lambda i: (i, 0)),
                  pl.BlockSpec((D, D), lambda i: (0, 0)),
                  pl.BlockSpec((1, D), lambda i: (0, 0)),
                  pl.BlockSpec((D, D), lambda i: (0, 0))],
        out_specs=pl.BlockSpec((BR, D), lambda i: (i, 0)),
        out_shape=jax.ShapeDtypeStruct((N, D), jnp.float32),
    )(summed, cnt, x_dst, Wl, bl.reshape(1, D), Wr)


def kernel(x_user, x_item, edge_index_user_to_item, edge_index_item_to_user,
           Wl1_u2i, bl1_u2i, Wr1_u2i, Wl1_i2u, bl1_i2u, Wr1_i2u,
           Wl2_u2i, bl2_u2i, Wr2_u2i, Wl2_i2u, bl2_i2u, Wr2_i2u):
    def prep(ei):
        src = ei[0].astype(jnp.int32)
        dst = ei[1].astype(jnp.int32)
        pad = E_PAD - src.shape[0]
        src2d = jnp.concatenate(
            [src, jnp.zeros((pad,), jnp.int32)]).reshape(E_ROWS, 128)
        dst2d = jnp.concatenate(
            [dst, jnp.full((pad,), 2 ** 30, jnp.int32)]).reshape(E_ROWS, 128)
        return src2d, dst2d

    su, du = prep(edge_index_user_to_item)
    si, di = prep(edge_index_item_to_user)
    zm = jnp.zeros((PER_TILE, D), jnp.float32)
    iden = jnp.arange(CROWS_PAD, dtype=jnp.int32).reshape(1, CROWS_PAD)

    def sage(x_src, x_dst, s2, d2, Wl, bl, Wr):
        sm, ct = _seg_sum(x_src, s2, d2, zm, iden)
        ct = ct.reshape(N_RANGES, CROWS_PAD, 128)[:, :CROWS, :]
        ct = ct.reshape(N_PAD)[:N]
        return _dense(sm[:N], ct.reshape(N, 1), x_dst, Wl, bl, Wr)

    item1 = sage(x_user, x_item, su, du, Wl1_u2i, bl1_u2i, Wr1_u2i)
    user1 = sage(x_item, x_user, si, di, Wl1_i2u, bl1_i2u, Wr1_i2u)
    item2 = sage(user1, item1, su, du, Wl2_u2i, bl2_u2i, Wr2_u2i)
    user2 = sage(item1, user1, si, di, Wl2_i2u, bl2_i2u, Wr2_i2u)
    return (user2, item2)


# speedup vs baseline: 5.7609x; 1.2349x over previous
import dataclasses
import functools

import jax
import jax.numpy as jnp
from jax import lax
from jax.experimental import pallas as pl
from jax.experimental.pallas import tpu as pltpu
from jax.experimental.pallas import tpu_sc as plsc

N = 50000
D = 128
E = 400000
E_ROWS = 3200
E_PAD = E_ROWS * 128
RANGE = 8448
PER_TILE = 528
N_RANGES = 6
N_PAD = N_RANGES * RANGE
CHUNK_ROWS = 8
TILE_EDGE_ROWS = E_ROWS // 16
CAPT = 12928
CAP_FLUSH = CAPT - 1024 - 128
NBUF = 2

CROWS_ALL = 400

_SC_PARAMS = pltpu.CompilerParams()
if "needs_layout_passes" in pltpu.CompilerParams.__dataclass_fields__:
    _SC_PARAMS = dataclasses.replace(_SC_PARAMS, needs_layout_passes=False)


def _count(dst_a, dst_b, zeros_mat, iden_row):
    @functools.partial(
        pl.kernel,
        out_type=jax.ShapeDtypeStruct((2 * CROWS_ALL, 128), jnp.float32),
        mesh=plsc.VectorSubcoreMesh(core_axis_name="c", subcore_axis_name="s"),
        compiler_params=_SC_PARAMS,
        scratch_types=[
            pltpu.VMEM((CHUNK_ROWS, 128), jnp.int32),
            pltpu.VMEM((CROWS_ALL, 128), jnp.float32),
            pltpu.VMEM((1, CROWS_ALL), jnp.int32),
            pltpu.VMEM_SHARED((CROWS_ALL, 128), jnp.float32),
        ],
    )
    def k(da_hbm, db_hbm, zm_hbm, iden_hbm, cnt_out, dst_v, cnt_v, iden_v,
          cacc):
        c = lax.axis_index("c")
        t = lax.axis_index("s")
        pltpu.sync_copy(iden_hbm, iden_v)
        pltpu.sync_copy(zm_hbm.at[pl.ds(0, CROWS_ALL), :], cnt_v)

        @pl.when(t == 0)
        def _():
            pltpu.sync_copy(zm_hbm.at[pl.ds(0, CROWS_ALL), :], cacc)

        plsc.subcore_barrier()
        ones16 = jnp.full((16,), 1.0, jnp.float32)

        def scan(d_hbm):
            def chunk_body(ci, carry):
                row = t * TILE_EDGE_ROWS + ci * CHUNK_ROWS
                pltpu.sync_copy(d_hbm.at[pl.ds(row, CHUNK_ROWS), :], dst_v)
                for r in range(CHUNK_ROWS):
                    for j in range(8):
                        d = dst_v[r, pl.ds(j * 16, 16)]
                        ldc = jnp.where(d < N_PAD, d, N_PAD)
                        plsc.addupdate_scatter(
                            cnt_v,
                            [lax.shift_right_logical(ldc, 7), ldc & 127],
                            ones16)
                return carry

            lax.fori_loop(0, TILE_EDGE_ROWS // CHUNK_ROWS, chunk_body,
                          jnp.int32(0))

        @pl.when(c == 0)
        def _():
            scan(da_hbm)

        @pl.when(c == 1)
        def _():
            scan(db_hbm)

        pltpu.sync_copy(cnt_v, cacc.at[iden_v.at[0]], add=True)
        plsc.subcore_barrier()

        @pl.when(t == 0)
        def _():
            pltpu.sync_copy(
                cacc, cnt_out.at[pl.ds(c * CROWS_ALL, CROWS_ALL), :])

        plsc.subcore_barrier()

    return k(dst_a, dst_b, zeros_mat, iden_row)


def _seg_sum(x_src, src2d, dst2d, zeros_mat):
    @functools.partial(
        pl.kernel,
        out_type=jax.ShapeDtypeStruct((N_PAD, D), jnp.float32),
        mesh=plsc.VectorSubcoreMesh(core_axis_name="c", subcore_axis_name="s"),
        compiler_params=_SC_PARAMS,
        scratch_types=[
            pltpu.VMEM((CHUNK_ROWS, 128), jnp.int32),
            pltpu.VMEM((CHUNK_ROWS, 128), jnp.int32),
            pltpu.VMEM((CAPT,), jnp.int32),
            pltpu.VMEM((CAPT,), jnp.int32),
            pltpu.VMEM((NBUF, 128), jnp.int32),
            pltpu.VMEM((NBUF, 128, D), jnp.float32),
            pltpu.VMEM_SHARED((RANGE + 1, D), jnp.float32),
            pltpu.SemaphoreType.DMA,
            pltpu.SemaphoreType.DMA,
        ],
    )
    def k(x_hbm, src_hbm, dst_hbm, zm_hbm,
          sum_out, src_v, dst_v, csrc, cdst, stage, rows_v, acc, sem0, sem1):
        sems = [sem0, sem1]
        c = lax.axis_index("c")
        t = lax.axis_index("s")
        tmask = jnp.full((16,), True)
        sent_d = jnp.full((16,), RANGE, jnp.int32)
        sent_s = jnp.zeros((16,), jnp.int32)
        for p in range(3):
            rid = 3 * c + p
            lo = rid * RANGE
            pltpu.sync_copy(zm_hbm, acc.at[pl.ds(t * PER_TILE, PER_TILE), :])
            plsc.subcore_barrier()

            def flush(nfl):
                def gs_body(b, carry):
                    for kk in range(8):
                        stage[0, pl.ds(kk * 16, 16)] = (
                            cdst[pl.ds(b * 128 + kk * 16, 16)])
                    pltpu.sync_copy(x_hbm.at[csrc.at[pl.ds(b * 128, 128)]],
                                    rows_v.at[0])
                    pltpu.sync_copy(rows_v.at[0], acc.at[stage.at[0]],
                                    add=True)
                    return carry

                lax.fori_loop(0, nfl, gs_body, jnp.int32(0))

            def chunk_body(ci, off):
                row = t * TILE_EDGE_ROWS + ci * CHUNK_ROWS
                pltpu.sync_copy(src_hbm.at[pl.ds(row, CHUNK_ROWS), :], src_v)
                pltpu.sync_copy(dst_hbm.at[pl.ds(row, CHUNK_ROWS), :], dst_v)
                for r in range(CHUNK_ROWS):
                    for j in range(8):
                        sl = pl.ds(j * 16, 16)
                        s = src_v[r, sl]
                        d = dst_v[r, sl]
                        ld = d - lo
                        ok = (ld >= 0) & (ld < RANGE)
                        plsc.store_compressed(csrc.at[pl.ds(off, 16)], s,
                                              mask=ok)
                        plsc.store_compressed(cdst.at[pl.ds(off, 16)], ld,
                                              mask=ok)
                        npc = plsc.all_reduce_population_count(ok)
                        off = off + lax.reduce_max(npc, (0,))

                do_flush = off >= CAP_FLUSH

                @pl.when(do_flush)
                def _():
                    nfl = lax.shift_right_logical(off, 7)
                    flush(nfl)
                    base = nfl * 128
                    for kk in range(8):
                        ts_ = csrc[pl.ds(base + kk * 16, 16)]
                        td_ = cdst[pl.ds(base + kk * 16, 16)]
                        csrc[pl.ds(kk * 16, 16)] = ts_
                        cdst[pl.ds(kk * 16, 16)] = td_

                return jnp.where(do_flush, off & 127, off)

            off = lax.fori_loop(0, TILE_EDGE_ROWS // CHUNK_ROWS, chunk_body,
                                jnp.int32(0))
            for kk in range(8):
                plsc.store_compressed(cdst.at[pl.ds(off + kk * 16, 16)],
                                      sent_d, mask=tmask)
                plsc.store_compressed(csrc.at[pl.ds(off + kk * 16, 16)],
                                      sent_s, mask=tmask)
            nblk = lax.shift_right_logical(off + 127, 7)

            for b in range(NBUF):
                @pl.when(b < nblk)
                def _(b=b):
                    pltpu.async_copy(
                        x_hbm.at[csrc.at[pl.ds(b * 128, 128)]],
                        rows_v.at[b], sems[b])

            def ring_body(go, carry):
                for b in range(NBUF):
                    g = go * NBUF + b

                    @pl.when(g < nblk)
                    def _(b=b, g=g):
                        pltpu.make_async_copy(
                            x_hbm.at[pl.ds(0, 128), :], rows_v.at[b],
                            sems[b]).wait()
                        for kk in range(8):
                            stage[b, pl.ds(kk * 16, 16)] = (
                                cdst[pl.ds(g * 128 + kk * 16, 16)])
                        pltpu.sync_copy(rows_v.at[b], acc.at[stage.at[b]],
                                        add=True)

                        @pl.when(g + NBUF < nblk)
                        def _():
                            pltpu.async_copy(
                                x_hbm.at[csrc.at[pl.ds((g + NBUF) * 128,
                                                       128)]],
                                rows_v.at[b], sems[b])
                return carry

            n_outer = (nblk + NBUF - 1) // NBUF
            lax.fori_loop(0, n_outer, ring_body, jnp.int32(0))

            plsc.subcore_barrier()
            pltpu.sync_copy(acc.at[pl.ds(t * PER_TILE, PER_TILE), :],
                            sum_out.at[pl.ds(lo + t * PER_TILE, PER_TILE), :])
            plsc.subcore_barrier()

    return k(x_src, src2d, dst2d, zeros_mat)


BR = 400


def _dense(summed, cnt, x_dst, Wl, bl, Wr):
    def body(s_ref, c_ref, xd_ref, wl_ref, bl_ref, wr_ref, o_ref):
        mean = s_ref[...] / jnp.maximum(c_ref[...], 1.0)
        out = (jnp.dot(mean, wl_ref[...], preferred_element_type=jnp.float32)
               + bl_ref[...]
               + jnp.dot(xd_ref[...], wr_ref[...],
                         preferred_element_type=jnp.float32))
        nrm = jnp.sqrt(jnp.sum(out * out, axis=1, keepdims=True))
        out = out / jnp.maximum(nrm, 1e-12)
        o_ref[...] = jnp.maximum(out, 0.0)

    return pl.pallas_call(
        body,
        grid=(N // BR,),
        in_specs=[pl.BlockSpec((BR, D), lambda i: (i, 0)),
                  pl.BlockSpec((BR, 1), lambda i: (i, 0)),
                  pl.BlockSpec((BR, D), lambda i: (i, 0)),
                  pl.BlockSpec((D, D), lambda i: (0, 0)),
                  pl.BlockSpec((1, D), lambda i: (0, 0)),
                  pl.BlockSpec((D, D), lambda i: (0, 0))],
        out_specs=pl.BlockSpec((BR, D), lambda i: (i, 0)),
        out_shape=jax.ShapeDtypeStruct((N, D), jnp.float32),
    )(summed, cnt, x_dst, Wl, bl.reshape(1, D), Wr)


def kernel(x_user, x_item, edge_index_user_to_item, edge_index_item_to_user,
           Wl1_u2i, bl1_u2i, Wr1_u2i, Wl1_i2u, bl1_i2u, Wr1_i2u,
           Wl2_u2i, bl2_u2i, Wr2_u2i, Wl2_i2u, bl2_i2u, Wr2_i2u):
    def prep(ei):
        src = ei[0].astype(jnp.int32)
        dst = ei[1].astype(jnp.int32)
        pad = E_PAD - src.shape[0]
        src2d = jnp.concatenate(
            [src, jnp.zeros((pad,), jnp.int32)]).reshape(E_ROWS, 128)
        dst2d = jnp.concatenate(
            [dst, jnp.full((pad,), 2 ** 30, jnp.int32)]).reshape(E_ROWS, 128)
        return src2d, dst2d

    su, du = prep(edge_index_user_to_item)
    si, di = prep(edge_index_item_to_user)
    zm = jnp.zeros((PER_TILE, D), jnp.float32)
    iden = jnp.arange(CROWS_ALL, dtype=jnp.int32).reshape(1, CROWS_ALL)

    cnts = _count(du, di, zm, iden)
    cnt_u2i = cnts[:CROWS_ALL].reshape(-1)[:N].reshape(N, 1)
    cnt_i2u = cnts[CROWS_ALL:].reshape(-1)[:N].reshape(N, 1)

    def sage(x_src, x_dst, s2, d2, cnt, Wl, bl, Wr):
        sm = _seg_sum(x_src, s2, d2, zm)
        return _dense(sm[:N], cnt, x_dst, Wl, bl, Wr)

    item1 = sage(x_user, x_item, su, du, cnt_u2i, Wl1_u2i, bl1_u2i, Wr1_u2i)
    user1 = sage(x_item, x_user, si, di, cnt_i2u, Wl1_i2u, bl1_i2u, Wr1_i2u)
    item2 = sage(user1, item1, su, du, cnt_u2i, Wl2_u2i, bl2_u2i, Wr2_u2i)
    user2 = sage(item1, user1, si, di, cnt_i2u, Wl2_i2u, bl2_i2u, Wr2_i2u)
    return (user2, item2)


# speedup vs baseline: 8.7608x; 1.5207x over previous
import dataclasses
import functools

import jax
import jax.numpy as jnp
from jax import lax
from jax.experimental import pallas as pl
from jax.experimental.pallas import tpu as pltpu
from jax.experimental.pallas import tpu_sc as plsc

N = 50000
D = 128
E = 400000
E_ROWS = 3200
E_PAD = E_ROWS * 128
RANGE = 8448
PER_TILE = 528
N_RANGES = 6
N_PAD = N_RANGES * RANGE
CHUNK_ROWS = 8
TILE_EDGE_ROWS = E_ROWS // 16
CAPT = 12416
CAP_FLUSH = CAPT - 1024 - 128
NBUF = 4
BLK = 64
NCHUNK = TILE_EDGE_ROWS // CHUNK_ROWS
SPLIT = 10

CROWS_ALL = 400

_SC_PARAMS = pltpu.CompilerParams()
if "needs_layout_passes" in pltpu.CompilerParams.__dataclass_fields__:
    _SC_PARAMS = dataclasses.replace(_SC_PARAMS, needs_layout_passes=False)


def _count(dst_a, dst_b, zeros_mat, iden_row):
    @functools.partial(
        pl.kernel,
        out_type=jax.ShapeDtypeStruct((2 * CROWS_ALL, 128), jnp.float32),
        mesh=plsc.VectorSubcoreMesh(core_axis_name="c", subcore_axis_name="s"),
        compiler_params=_SC_PARAMS,
        scratch_types=[
            pltpu.VMEM((CHUNK_ROWS, 128), jnp.int32),
            pltpu.VMEM((CROWS_ALL, 128), jnp.float32),
            pltpu.VMEM((1, CROWS_ALL), jnp.int32),
            pltpu.VMEM_SHARED((CROWS_ALL, 128), jnp.float32),
        ],
    )
    def k(da_hbm, db_hbm, zm_hbm, iden_hbm, cnt_out, dst_v, cnt_v, iden_v,
          cacc):
        c = lax.axis_index("c")
        t = lax.axis_index("s")
        pltpu.sync_copy(iden_hbm, iden_v)
        pltpu.sync_copy(zm_hbm.at[pl.ds(0, CROWS_ALL), :], cnt_v)

        @pl.when(t == 0)
        def _():
            pltpu.sync_copy(zm_hbm.at[pl.ds(0, CROWS_ALL), :], cacc)

        plsc.subcore_barrier()
        ones16 = jnp.full((16,), 1.0, jnp.float32)

        def scan(d_hbm):
            def chunk_body(ci, carry):
                row = t * TILE_EDGE_ROWS + ci * CHUNK_ROWS
                pltpu.sync_copy(d_hbm.at[pl.ds(row, CHUNK_ROWS), :], dst_v)
                for r in range(CHUNK_ROWS):
                    for j in range(8):
                        d = dst_v[r, pl.ds(j * 16, 16)]
                        ldc = jnp.where(d < N_PAD, d, N_PAD)
                        plsc.addupdate_scatter(
                            cnt_v,
                            [lax.shift_right_logical(ldc, 7), ldc & 127],
                            ones16)
                return carry

            lax.fori_loop(0, TILE_EDGE_ROWS // CHUNK_ROWS, chunk_body,
                          jnp.int32(0))

        @pl.when(c == 0)
        def _():
            scan(da_hbm)

        @pl.when(c == 1)
        def _():
            scan(db_hbm)

        pltpu.sync_copy(cnt_v, cacc.at[iden_v.at[0]], add=True)
        plsc.subcore_barrier()

        @pl.when(t == 0)
        def _():
            pltpu.sync_copy(
                cacc, cnt_out.at[pl.ds(c * CROWS_ALL, CROWS_ALL), :])

        plsc.subcore_barrier()

    return k(dst_a, dst_b, zeros_mat, iden_row)


def _seg_sum(x_src, src2d, dst2d, zeros_mat):
    @functools.partial(
        pl.kernel,
        out_type=jax.ShapeDtypeStruct((N_PAD, D), jnp.float32),
        mesh=plsc.VectorSubcoreMesh(core_axis_name="c", subcore_axis_name="s"),
        compiler_params=_SC_PARAMS,
        scratch_types=[
            pltpu.VMEM((2, CHUNK_ROWS, 128), jnp.int32),
            pltpu.VMEM((2, CHUNK_ROWS, 128), jnp.int32),
            pltpu.VMEM((CAPT,), jnp.int32),
            pltpu.VMEM((CAPT,), jnp.int32),
            pltpu.VMEM((NBUF, BLK), jnp.int32),
            pltpu.VMEM((NBUF, BLK, D), jnp.float32),
            pltpu.VMEM_SHARED((RANGE + 1, D), jnp.float32),
            pltpu.SemaphoreType.DMA,
            pltpu.SemaphoreType.DMA,
            pltpu.SemaphoreType.DMA,
            pltpu.SemaphoreType.DMA,
            pltpu.SemaphoreType.DMA,
            pltpu.SemaphoreType.DMA,
            pltpu.SemaphoreType.DMA,
        ],
    )
    def k(x_hbm, src_hbm, dst_hbm, zm_hbm,
          sum_out, src_v, dst_v, csrc, cdst, stage, rows_v, acc,
          semi0, semi1, semz, semg0, semg1, semg2, semg3):
        semi = [semi0, semi1]
        semg = [semg0, semg1, semg2, semg3]
        c = lax.axis_index("c")
        t = lax.axis_index("s")
        tmask = jnp.full((16,), True)
        sent_d = jnp.full((16,), RANGE, jnp.int32)
        sent_s = jnp.zeros((16,), jnp.int32)
        for p in range(3):
            rid = 3 * c + p
            lo = rid * RANGE
            tile0 = t * TILE_EDGE_ROWS
            pltpu.async_copy(
                zm_hbm, acc.at[pl.ds(t * PER_TILE, PER_TILE), :], semz)
            pltpu.async_copy(
                src_hbm.at[pl.ds(tile0, CHUNK_ROWS), :], src_v.at[0], semi[0])
            pltpu.async_copy(
                dst_hbm.at[pl.ds(tile0, CHUNK_ROWS), :], dst_v.at[0], semi[0])

            def flush(nfl):
                def gs_body(b, carry):
                    for kk in range(BLK // 16):
                        stage[0, pl.ds(kk * 16, 16)] = (
                            cdst[pl.ds(b * BLK + kk * 16, 16)])
                    pltpu.sync_copy(x_hbm.at[csrc.at[pl.ds(b * BLK, BLK)]],
                                    rows_v.at[0])
                    pltpu.sync_copy(rows_v.at[0], acc.at[stage.at[0]],
                                    add=True)
                    return carry

                lax.fori_loop(0, nfl, gs_body, jnp.int32(0))

            def chunk_body(ci, off):
                bk = ci & 1
                nrow = tile0 + (ci + 1) * CHUNK_ROWS
                for bb in range(2):
                    @pl.when(bk == bb)
                    def _(bb=bb):
                        pltpu.make_async_copy(
                            src_hbm.at[pl.ds(0, CHUNK_ROWS), :],
                            src_v.at[bb], semi[bb]).wait()
                        pltpu.make_async_copy(
                            src_hbm.at[pl.ds(0, CHUNK_ROWS), :],
                            dst_v.at[bb], semi[bb]).wait()

                        @pl.when(ci + 1 < NCHUNK)
                        def _():
                            pltpu.async_copy(
                                src_hbm.at[pl.ds(nrow, CHUNK_ROWS), :],
                                src_v.at[1 - bb], semi[1 - bb])
                            pltpu.async_copy(
                                dst_hbm.at[pl.ds(nrow, CHUNK_ROWS), :],
                                dst_v.at[1 - bb], semi[1 - bb])

                for r in range(CHUNK_ROWS):
                    for j in range(8):
                        sl = pl.ds(j * 16, 16)
                        s = src_v[bk, r, sl]
                        d = dst_v[bk, r, sl]
                        ld = d - lo
                        ok = (ld >= 0) & (ld < RANGE)
                        plsc.store_compressed(csrc.at[pl.ds(off, 16)], s,
                                              mask=ok)
                        plsc.store_compressed(cdst.at[pl.ds(off, 16)], ld,
                                              mask=ok)
                        npc = plsc.all_reduce_population_count(ok)
                        off = off + lax.reduce_max(npc, (0,))

                do_flush = off >= CAP_FLUSH

                @pl.when(do_flush)
                def _():
                    nfl = lax.shift_right_logical(off, 6)
                    flush(nfl)
                    base = nfl * BLK
                    for kk in range(BLK // 16):
                        ts_ = csrc[pl.ds(base + kk * 16, 16)]
                        td_ = cdst[pl.ds(base + kk * 16, 16)]
                        csrc[pl.ds(kk * 16, 16)] = ts_
                        cdst[pl.ds(kk * 16, 16)] = td_

                return jnp.where(do_flush, off & (BLK - 1), off)

            off = lax.fori_loop(0, SPLIT, chunk_body, jnp.int32(0))
            pltpu.make_async_copy(
                zm_hbm, acc.at[pl.ds(t * PER_TILE, PER_TILE), :],
                semz).wait()
            plsc.subcore_barrier()
            off = lax.fori_loop(SPLIT, NCHUNK, chunk_body, off)

            for kk in range(8):
                plsc.store_compressed(cdst.at[pl.ds(off + kk * 16, 16)],
                                      sent_d, mask=tmask)
                plsc.store_compressed(csrc.at[pl.ds(off + kk * 16, 16)],
                                      sent_s, mask=tmask)
            nblk = lax.shift_right_logical(off + BLK - 1, 6)

            for b in range(NBUF):
                @pl.when(b < nblk)
                def _(b=b):
                    pltpu.async_copy(
                        x_hbm.at[csrc.at[pl.ds(b * BLK, BLK)]],
                        rows_v.at[b], semg[b])

            def ring_body(go, carry):
                for b in range(NBUF):
                    g = go * NBUF + b

                    @pl.when(g < nblk)
                    def _(b=b, g=g):
                        pltpu.make_async_copy(
                            x_hbm.at[pl.ds(0, BLK), :], rows_v.at[b],
                            semg[b]).wait()
                        for kk in range(BLK // 16):
                            stage[b, pl.ds(kk * 16, 16)] = (
                                cdst[pl.ds(g * BLK + kk * 16, 16)])
                        pltpu.sync_copy(rows_v.at[b], acc.at[stage.at[b]],
                                        add=True)

                        @pl.when(g + NBUF < nblk)
                        def _():
                            pltpu.async_copy(
                                x_hbm.at[csrc.at[pl.ds((g + NBUF) * BLK,
                                                       BLK)]],
                                rows_v.at[b], semg[b])
                return carry

            n_outer = (nblk + NBUF - 1) // NBUF
            lax.fori_loop(0, n_outer, ring_body, jnp.int32(0))

            plsc.subcore_barrier()
            pltpu.sync_copy(acc.at[pl.ds(t * PER_TILE, PER_TILE), :],
                            sum_out.at[pl.ds(lo + t * PER_TILE, PER_TILE), :])
            plsc.subcore_barrier()

    return k(x_src, src2d, dst2d, zeros_mat)


BR = 400


def _dense(summed, cnt, x_dst, Wl, bl, Wr):
    def body(s_ref, c_ref, xd_ref, wl_ref, bl_ref, wr_ref, o_ref):
        mean = s_ref[...] / jnp.maximum(c_ref[...], 1.0)
        out = (jnp.dot(mean, wl_ref[...], preferred_element_type=jnp.float32)
               + bl_ref[...]
               + jnp.dot(xd_ref[...], wr_ref[...],
                         preferred_element_type=jnp.float32))
        nrm = jnp.sqrt(jnp.sum(out * out, axis=1, keepdims=True))
        out = out / jnp.maximum(nrm, 1e-12)
        o_ref[...] = jnp.maximum(out, 0.0)

    return pl.pallas_call(
        body,
        grid=(N // BR,),
        in_specs=[pl.BlockSpec((BR, D), lambda i: (i, 0)),
                  pl.BlockSpec((BR, 1), lambda i: (i, 0)),
                  pl.BlockSpec((BR, D), lambda i: (i, 0)),
                  pl.BlockSpec((D, D), lambda i: (0, 0)),
                  pl.BlockSpec((1, D), lambda i: (0, 0)),
                  pl.BlockSpec((D, D), lambda i: (0, 0))],
        out_specs=pl.BlockSpec((BR, D), lambda i: (i, 0)),
        out_shape=jax.ShapeDtypeStruct((N, D), jnp.float32),
    )(summed, cnt, x_dst, Wl, bl.reshape(1, D), Wr)


def kernel(x_user, x_item, edge_index_user_to_item, edge_index_item_to_user,
           Wl1_u2i, bl1_u2i, Wr1_u2i, Wl1_i2u, bl1_i2u, Wr1_i2u,
           Wl2_u2i, bl2_u2i, Wr2_u2i, Wl2_i2u, bl2_i2u, Wr2_i2u):
    def prep(ei):
        src = ei[0].astype(jnp.int32)
        dst = ei[1].astype(jnp.int32)
        pad = E_PAD - src.shape[0]
        src2d = jnp.concatenate(
            [src, jnp.zeros((pad,), jnp.int32)]).reshape(E_ROWS, 128)
        dst2d = jnp.concatenate(
            [dst, jnp.full((pad,), 2 ** 30, jnp.int32)]).reshape(E_ROWS, 128)
        return src2d, dst2d

    su, du = prep(edge_index_user_to_item)
    si, di = prep(edge_index_item_to_user)
    zm = jnp.zeros((PER_TILE, D), jnp.float32)
    iden = jnp.arange(CROWS_ALL, dtype=jnp.int32).reshape(1, CROWS_ALL)

    cnts = _count(du, di, zm, iden)
    cnt_u2i = cnts[:CROWS_ALL].reshape(-1)[:N].reshape(N, 1)
    cnt_i2u = cnts[CROWS_ALL:].reshape(-1)[:N].reshape(N, 1)

    def sage(x_src, x_dst, s2, d2, cnt, Wl, bl, Wr):
        sm = _seg_sum(x_src, s2, d2, zm)
        return _dense(sm[:N], cnt, x_dst, Wl, bl, Wr)

    item1 = sage(x_user, x_item, su, du, cnt_u2i, Wl1_u2i, bl1_u2i, Wr1_u2i)
    user1 = sage(x_item, x_user, si, di, cnt_i2u, Wl1_i2u, bl1_i2u, Wr1_i2u)
    item2 = sage(user1, item1, su, du, cnt_u2i, Wl2_u2i, bl2_u2i, Wr2_u2i)
    user2 = sage(item1, user1, si, di, cnt_i2u, Wl2_i2u, bl2_i2u, Wr2_i2u)
    return (user2, item2)


# speedup vs baseline: 8.8046x; 1.0050x over previous
import dataclasses
import functools

import jax
import jax.numpy as jnp
from jax import lax
from jax.experimental import pallas as pl
from jax.experimental.pallas import tpu as pltpu
from jax.experimental.pallas import tpu_sc as plsc

N = 50000
D = 128
E = 400000
E_ROWS = 3200
E_PAD = E_ROWS * 128
RANGE = 8448
PER_TILE = 528
N_RANGES = 6
N_PAD = N_RANGES * RANGE
CHUNK_ROWS = 8
TILE_EDGE_ROWS = E_ROWS // 16
CAPT = 8576
CAP_FLUSH = CAPT - 1024 - 128
NBUF = 5
BLK = 64
LOGBLK = 6
NCHUNK = TILE_EDGE_ROWS // CHUNK_ROWS
SPLIT = 7

CROWS_ALL = 400

_SC_PARAMS = pltpu.CompilerParams()
if "needs_layout_passes" in pltpu.CompilerParams.__dataclass_fields__:
    _SC_PARAMS = dataclasses.replace(_SC_PARAMS, needs_layout_passes=False)


def _count(dst_a, dst_b, zeros_mat, iden_row):
    @functools.partial(
        pl.kernel,
        out_type=jax.ShapeDtypeStruct((2 * CROWS_ALL, 128), jnp.float32),
        mesh=plsc.VectorSubcoreMesh(core_axis_name="c", subcore_axis_name="s"),
        compiler_params=_SC_PARAMS,
        scratch_types=[
            pltpu.VMEM((CHUNK_ROWS, 128), jnp.int32),
            pltpu.VMEM((CROWS_ALL, 128), jnp.float32),
            pltpu.VMEM((1, CROWS_ALL), jnp.int32),
            pltpu.VMEM_SHARED((CROWS_ALL, 128), jnp.float32),
        ],
    )
    def k(da_hbm, db_hbm, zm_hbm, iden_hbm, cnt_out, dst_v, cnt_v, iden_v,
          cacc):
        c = lax.axis_index("c")
        t = lax.axis_index("s")
        pltpu.sync_copy(iden_hbm, iden_v)
        pltpu.sync_copy(zm_hbm.at[pl.ds(0, CROWS_ALL), :], cnt_v)

        @pl.when(t == 0)
        def _():
            pltpu.sync_copy(zm_hbm.at[pl.ds(0, CROWS_ALL), :], cacc)

        plsc.subcore_barrier()
        ones16 = jnp.full((16,), 1.0, jnp.float32)

        def scan(d_hbm):
            def chunk_body(ci, carry):
                row = t * TILE_EDGE_ROWS + ci * CHUNK_ROWS
                pltpu.sync_copy(d_hbm.at[pl.ds(row, CHUNK_ROWS), :], dst_v)
                for r in range(CHUNK_ROWS):
                    for j in range(8):
                        d = dst_v[r, pl.ds(j * 16, 16)]
                        ldc = jnp.where(d < N_PAD, d, N_PAD)
                        plsc.addupdate_scatter(
                            cnt_v,
                            [lax.shift_right_logical(ldc, 7), ldc & 127],
                            ones16)
                return carry

            lax.fori_loop(0, TILE_EDGE_ROWS // CHUNK_ROWS, chunk_body,
                          jnp.int32(0))

        @pl.when(c == 0)
        def _():
            scan(da_hbm)

        @pl.when(c == 1)
        def _():
            scan(db_hbm)

        pltpu.sync_copy(cnt_v, cacc.at[iden_v.at[0]], add=True)
        plsc.subcore_barrier()

        @pl.when(t == 0)
        def _():
            pltpu.sync_copy(
                cacc, cnt_out.at[pl.ds(c * CROWS_ALL, CROWS_ALL), :])

        plsc.subcore_barrier()

    return k(dst_a, dst_b, zeros_mat, iden_row)


def _seg_sum(x_src, src2d, dst2d, zeros_mat):
    @functools.partial(
        pl.kernel,
        out_type=jax.ShapeDtypeStruct((N_PAD, D), jnp.float32),
        mesh=plsc.VectorSubcoreMesh(core_axis_name="c", subcore_axis_name="s"),
        compiler_params=_SC_PARAMS,
        scratch_types=[
            pltpu.VMEM((2, CHUNK_ROWS, 128), jnp.int32),
            pltpu.VMEM((2, CHUNK_ROWS, 128), jnp.int32),
            pltpu.VMEM((CAPT,), jnp.int32),
            pltpu.VMEM((CAPT,), jnp.int32),
            pltpu.VMEM((NBUF, BLK), jnp.int32),
            pltpu.VMEM((NBUF, BLK, D), jnp.float32),
            pltpu.VMEM_SHARED((RANGE + 1, D), jnp.float32),
            pltpu.SemaphoreType.DMA,
            pltpu.SemaphoreType.DMA,
            pltpu.SemaphoreType.DMA,
            pltpu.SemaphoreType.DMA,
            pltpu.SemaphoreType.DMA,
            pltpu.SemaphoreType.DMA,
            pltpu.SemaphoreType.DMA,
            pltpu.SemaphoreType.DMA,
        ],
    )
    def k(x_hbm, src_hbm, dst_hbm, zm_hbm,
          sum_out, src_v, dst_v, csrc, cdst, stage, rows_v, acc,
          semi0, semi1, semz, semg0, semg1, semg2, semg3, semg4):
        semi = [semi0, semi1]
        semg = [semg0, semg1, semg2, semg3, semg4]
        c = lax.axis_index("c")
        t = lax.axis_index("s")
        tmask = jnp.full((16,), True)
        sent_d = jnp.full((16,), RANGE, jnp.int32)
        sent_s = jnp.zeros((16,), jnp.int32)
        for p in range(3):
            rid = 3 * c + p
            lo = rid * RANGE
            tile0 = t * TILE_EDGE_ROWS
            pltpu.async_copy(
                zm_hbm, acc.at[pl.ds(t * PER_TILE, PER_TILE), :], semz)
            pltpu.async_copy(
                src_hbm.at[pl.ds(tile0, CHUNK_ROWS), :], src_v.at[0], semi[0])
            pltpu.async_copy(
                dst_hbm.at[pl.ds(tile0, CHUNK_ROWS), :], dst_v.at[0], semi[0])

            def flush(nfl):
                def gs_body(b, carry):
                    for kk in range(BLK // 16):
                        stage[0, pl.ds(kk * 16, 16)] = (
                            cdst[pl.ds(b * BLK + kk * 16, 16)])
                    pltpu.sync_copy(x_hbm.at[csrc.at[pl.ds(b * BLK, BLK)]],
                                    rows_v.at[0])
                    pltpu.sync_copy(rows_v.at[0], acc.at[stage.at[0]],
                                    add=True)
                    return carry

                lax.fori_loop(0, nfl, gs_body, jnp.int32(0))

            def chunk_body(ci, off):
                bk = ci & 1
                nrow = tile0 + (ci + 1) * CHUNK_ROWS
                for bb in range(2):
                    @pl.when(bk == bb)
                    def _(bb=bb):
                        pltpu.make_async_copy(
                            src_hbm.at[pl.ds(0, CHUNK_ROWS), :],
                            src_v.at[bb], semi[bb]).wait()
                        pltpu.make_async_copy(
                            src_hbm.at[pl.ds(0, CHUNK_ROWS), :],
                            dst_v.at[bb], semi[bb]).wait()

                        @pl.when(ci + 1 < NCHUNK)
                        def _():
                            pltpu.async_copy(
                                src_hbm.at[pl.ds(nrow, CHUNK_ROWS), :],
                                src_v.at[1 - bb], semi[1 - bb])
                            pltpu.async_copy(
                                dst_hbm.at[pl.ds(nrow, CHUNK_ROWS), :],
                                dst_v.at[1 - bb], semi[1 - bb])

                for r in range(CHUNK_ROWS):
                    for j in range(8):
                        sl = pl.ds(j * 16, 16)
                        s = src_v[bk, r, sl]
                        d = dst_v[bk, r, sl]
                        ld = d - lo
                        ok = (ld >= 0) & (ld < RANGE)
                        plsc.store_compressed(csrc.at[pl.ds(off, 16)], s,
                                              mask=ok)
                        plsc.store_compressed(cdst.at[pl.ds(off, 16)], ld,
                                              mask=ok)
                        npc = plsc.all_reduce_population_count(ok)
                        off = off + lax.reduce_max(npc, (0,))

                do_flush = off >= CAP_FLUSH

                @pl.when(do_flush)
                def _():
                    nfl = lax.shift_right_logical(off, LOGBLK)
                    flush(nfl)
                    base = nfl * BLK
                    for kk in range(BLK // 16):
                        ts_ = csrc[pl.ds(base + kk * 16, 16)]
                        td_ = cdst[pl.ds(base + kk * 16, 16)]
                        csrc[pl.ds(kk * 16, 16)] = ts_
                        cdst[pl.ds(kk * 16, 16)] = td_

                return jnp.where(do_flush, off & (BLK - 1), off)

            off = lax.fori_loop(0, SPLIT, chunk_body, jnp.int32(0))
            pltpu.make_async_copy(
                zm_hbm, acc.at[pl.ds(t * PER_TILE, PER_TILE), :],
                semz).wait()
            plsc.subcore_barrier()
            off = lax.fori_loop(SPLIT, NCHUNK, chunk_body, off)

            for kk in range(8):
                plsc.store_compressed(cdst.at[pl.ds(off + kk * 16, 16)],
                                      sent_d, mask=tmask)
                plsc.store_compressed(csrc.at[pl.ds(off + kk * 16, 16)],
                                      sent_s, mask=tmask)
            nblk = lax.shift_right_logical(off + BLK - 1, LOGBLK)

            for b in range(NBUF):
                @pl.when(b < nblk)
                def _(b=b):
                    pltpu.async_copy(
                        x_hbm.at[csrc.at[pl.ds(b * BLK, BLK)]],
                        rows_v.at[b], semg[b])

            def ring_body(go, carry):
                for b in range(NBUF):
                    g = go * NBUF + b

                    @pl.when(g < nblk)
                    def _(b=b, g=g):
                        pltpu.make_async_copy(
                            x_hbm.at[pl.ds(0, BLK), :], rows_v.at[b],
                            semg[b]).wait()
                        for kk in range(BLK // 16):
                            stage[b, pl.ds(kk * 16, 16)] = (
                                cdst[pl.ds(g * BLK + kk * 16, 16)])
                        pltpu.sync_copy(rows_v.at[b], acc.at[stage.at[b]],
                                        add=True)

                        @pl.when(g + NBUF < nblk)
                        def _():
                            pltpu.async_copy(
                                x_hbm.at[csrc.at[pl.ds((g + NBUF) * BLK,
                                                       BLK)]],
                                rows_v.at[b], semg[b])
                return carry

            n_outer = (nblk + NBUF - 1) // NBUF
            lax.fori_loop(0, n_outer, ring_body, jnp.int32(0))

            plsc.subcore_barrier()
            pltpu.sync_copy(acc.at[pl.ds(t * PER_TILE, PER_TILE), :],
                            sum_out.at[pl.ds(lo + t * PER_TILE, PER_TILE), :])
            plsc.subcore_barrier()

    return k(x_src, src2d, dst2d, zeros_mat)


BR = 400


def _dense(summed, cnt, x_dst, Wl, bl, Wr):
    def body(s_ref, c_ref, xd_ref, wl_ref, bl_ref, wr_ref, o_ref):
        mean = s_ref[...] / jnp.maximum(c_ref[...], 1.0)
        out = (jnp.dot(mean, wl_ref[...], preferred_element_type=jnp.float32)
               + bl_ref[...]
               + jnp.dot(xd_ref[...], wr_ref[...],
                         preferred_element_type=jnp.float32))
        nrm = jnp.sqrt(jnp.sum(out * out, axis=1, keepdims=True))
        out = out / jnp.maximum(nrm, 1e-12)
        o_ref[...] = jnp.maximum(out, 0.0)

    return pl.pallas_call(
        body,
        grid=(N // BR,),
        in_specs=[pl.BlockSpec((BR, D), lambda i: (i, 0)),
                  pl.BlockSpec((BR, 1), lambda i: (i, 0)),
                  pl.BlockSpec((BR, D), lambda i: (i, 0)),
                  pl.BlockSpec((D, D), lambda i: (0, 0)),
                  pl.BlockSpec((1, D), lambda i: (0, 0)),
                  pl.BlockSpec((D, D), lambda i: (0, 0))],
        out_specs=pl.BlockSpec((BR, D), lambda i: (i, 0)),
        out_shape=jax.ShapeDtypeStruct((N, D), jnp.float32),
    )(summed, cnt, x_dst, Wl, bl.reshape(1, D), Wr)


def kernel(x_user, x_item, edge_index_user_to_item, edge_index_item_to_user,
           Wl1_u2i, bl1_u2i, Wr1_u2i, Wl1_i2u, bl1_i2u, Wr1_i2u,
           Wl2_u2i, bl2_u2i, Wr2_u2i, Wl2_i2u, bl2_i2u, Wr2_i2u):
    def prep(ei):
        src = ei[0].astype(jnp.int32)
        dst = ei[1].astype(jnp.int32)
        pad = E_PAD - src.shape[0]
        src2d = jnp.concatenate(
            [src, jnp.zeros((pad,), jnp.int32)]).reshape(E_ROWS, 128)
        dst2d = jnp.concatenate(
            [dst, jnp.full((pad,), 2 ** 30, jnp.int32)]).reshape(E_ROWS, 128)
        return src2d, dst2d

    su, du = prep(edge_index_user_to_item)
    si, di = prep(edge_index_item_to_user)
    zm = jnp.zeros((PER_TILE, D), jnp.float32)
    iden = jnp.arange(CROWS_ALL, dtype=jnp.int32).reshape(1, CROWS_ALL)

    cnts = _count(du, di, zm, iden)
    cnt_u2i = cnts[:CROWS_ALL].reshape(-1)[:N].reshape(N, 1)
    cnt_i2u = cnts[CROWS_ALL:].reshape(-1)[:N].reshape(N, 1)

    def sage(x_src, x_dst, s2, d2, cnt, Wl, bl, Wr):
        sm = _seg_sum(x_src, s2, d2, zm)
        return _dense(sm[:N], cnt, x_dst, Wl, bl, Wr)

    item1 = sage(x_user, x_item, su, du, cnt_u2i, Wl1_u2i, bl1_u2i, Wr1_u2i)
    user1 = sage(x_item, x_user, si, di, cnt_i2u, Wl1_i2u, bl1_i2u, Wr1_i2u)
    item2 = sage(user1, item1, su, du, cnt_u2i, Wl2_u2i, bl2_u2i, Wr2_u2i)
    user2 = sage(item1, user1, si, di, cnt_i2u, Wl2_i2u, bl2_i2u, Wr2_i2u)
    return (user2, item2)
